# Initial kernel scaffold; baseline (speedup 1.0000x reference)
import jax
import jax.numpy as jnp
from jax.experimental import pallas as pl


def kernel(x, edge_index, batch, W_enc, b_enc, weight, W_ih, W_hh, b_ih, b_hh, W1, b1, W2, b2, W3, b3):
    raise NotImplementedError("write your pallas kernel here")


# speedup vs baseline: 2.8834x; 2.8834x over previous
import functools

import jax
import jax.numpy as jnp
from jax import lax
from jax.experimental import pallas as pl
from jax.experimental.pallas import tpu as pltpu
from jax.experimental.pallas import tpu_sc as plsc

_N = 10000
_E = 320000
_DIN = 128
_C = 101
_OUT = 200
_L = 6
_G = 256
_F = 256
_H = _F // 2
_BLK = 1000
_K = 80
_EPT = _E // 16
_CHUNKS = _EPT // _K
_RPT = _N // 16
_ZR = 25


_sc_mesh = plsc.VectorSubcoreMesh(core_axis_name="c", subcore_axis_name="s")


@functools.partial(
    pl.kernel,
    out_type=jax.ShapeDtypeStruct((2, _N, _H), jnp.float32),
    mesh=_sc_mesh,
    compiler_params=pltpu.CompilerParams(use_tc_tiling_on_sc=False),
    scratch_types=[
        pltpu.VMEM((_K,), jnp.int32),
        pltpu.VMEM((_K,), jnp.int32),
        pltpu.VMEM((_K,), jnp.int32),
        pltpu.VMEM((_K, _H), jnp.float32),
        pltpu.VMEM((_ZR, _H), jnp.float32),
        pltpu.VMEM_SHARED((_N, _H), jnp.float32),
        pltpu.SemaphoreType.DMA,
    ],
)
def _segsum(m_hbm, src_hbm, dst_hbm, zc_hbm, out_hbm,
            src_v, src2_v, dst_v, rows_v, zbuf, agg_sh, sem):
    cid = lax.axis_index("c")
    sid = lax.axis_index("s")
    pltpu.sync_copy(zc_hbm, zbuf)
    r0 = sid * _RPT

    def _zero(k, carry):
        pltpu.sync_copy(zbuf, agg_sh.at[pl.ds(r0 + k * _ZR, _ZR)])
        return carry

    lax.fori_loop(0, _RPT // _ZR, _zero, 0)
    plsc.subcore_barrier()

    base_e = sid * _EPT
    off = cid * _N

    def _edges(j, carry):
        e0 = base_e + j * _K
        pltpu.sync_copy(src_hbm.at[pl.ds(e0, _K)], src_v)
        pltpu.sync_copy(dst_hbm.at[pl.ds(e0, _K)], dst_v)

        def _adj(t, c):
            src2_v[pl.ds(t * 16, 16)] = src_v[pl.ds(t * 16, 16)] + off
            return c

        lax.fori_loop(0, _K // 16, _adj, 0)
        pltpu.async_copy(m_hbm.at[src2_v], rows_v, sem).wait()
        pltpu.sync_copy(rows_v, agg_sh.at[dst_v], add=True)
        return carry

    lax.fori_loop(0, _CHUNKS, _edges, 0)
    plsc.subcore_barrier()

    def _out(k, carry):
        rr = r0 + k * _ZR
        pltpu.sync_copy(agg_sh.at[pl.ds(rr, _ZR)],
                        out_hbm.at[cid, pl.ds(rr, _ZR), :])
        return carry

    lax.fori_loop(0, _RPT // _ZR, _out, 0)


def _encode_body(x_ref, we_ref, be_ref, w0_ref, h_ref, m0_ref):
    h = jnp.maximum(
        jnp.dot(x_ref[...], we_ref[...], preferred_element_type=jnp.float32)
        + be_ref[...], 0.0)
    h_ref[...] = h
    m = jnp.dot(h, w0_ref[...], preferred_element_type=jnp.float32)
    m0_ref[0] = m[:, :_H]
    m0_ref[1] = m[:, _H:]


_encode = pl.pallas_call(
    _encode_body,
    grid=(_N // _BLK,),
    in_specs=[
        pl.BlockSpec((_BLK, _DIN), lambda i: (i, 0)),
        pl.BlockSpec((_DIN, _OUT), lambda i: (0, 0)),
        pl.BlockSpec((1, _OUT), lambda i: (0, 0)),
        pl.BlockSpec((_OUT, _F), lambda i: (0, 0)),
    ],
    out_specs=[
        pl.BlockSpec((_BLK, _OUT), lambda i: (i, 0)),
        pl.BlockSpec((2, _BLK, _H), lambda i: (0, i, 0)),
    ],
    out_shape=[
        jax.ShapeDtypeStruct((_N, _OUT), jnp.float32),
        jax.ShapeDtypeStruct((2, _N, _H), jnp.float32),
    ],
)


def _gru_body(a_ref, h_ref, wir, wiz, win, whr, whz, whn,
              bir, biz, bin_, bhr, bhz, bhn, wnx, ho_ref, mo_ref):
    av = a_ref[...]
    a = jnp.concatenate([av[0], av[1]], axis=1)
    h = h_ref[...]
    f32 = jnp.float32
    i_r = jnp.dot(a, wir[...], preferred_element_type=f32) + bir[...]
    i_z = jnp.dot(a, wiz[...], preferred_element_type=f32) + biz[...]
    i_n = jnp.dot(a, win[...], preferred_element_type=f32) + bin_[...]
    h_r = jnp.dot(h, whr[...], preferred_element_type=f32) + bhr[...]
    h_z = jnp.dot(h, whz[...], preferred_element_type=f32) + bhz[...]
    h_n = jnp.dot(h, whn[...], preferred_element_type=f32) + bhn[...]
    r = jax.nn.sigmoid(i_r + h_r)
    z = jax.nn.sigmoid(i_z + h_z)
    n = jnp.tanh(i_n + r * h_n)
    hn = (1.0 - z) * n + z * h
    ho_ref[...] = hn
    mn = jnp.dot(hn, wnx[...], preferred_element_type=f32)
    mo_ref[0] = mn[:, :_H]
    mo_ref[1] = mn[:, _H:]


_gru = pl.pallas_call(
    _gru_body,
    grid=(_N // _BLK,),
    in_specs=[
        pl.BlockSpec((2, _BLK, _H), lambda i: (0, i, 0)),
        pl.BlockSpec((_BLK, _OUT), lambda i: (i, 0)),
    ] + [pl.BlockSpec((_F, _OUT), lambda i: (0, 0))] * 3
      + [pl.BlockSpec((_OUT, _OUT), lambda i: (0, 0))] * 3
      + [pl.BlockSpec((1, _OUT), lambda i: (0, 0))] * 6
      + [pl.BlockSpec((_OUT, _F), lambda i: (0, 0))],
    out_specs=[
        pl.BlockSpec((_BLK, _OUT), lambda i: (i, 0)),
        pl.BlockSpec((2, _BLK, _H), lambda i: (0, i, 0)),
    ],
    out_shape=[
        jax.ShapeDtypeStruct((_N, _OUT), jnp.float32),
        jax.ShapeDtypeStruct((2, _N, _H), jnp.float32),
    ],
)


def _pool_body(h_ref, hc_ref, b_ref, w1a, w1b, b1, w2, b2, w3, b3, o_ref):
    f32 = jnp.float32
    bvec = b_ref[...]
    gids = lax.broadcasted_iota(jnp.int32, (_G, _N), 0)
    onehot = jnp.where(bvec == gids, 1.0, 0.0).astype(f32)
    cnt = jnp.sum(onehot, axis=1, keepdims=True)
    inv = 1.0 / jnp.maximum(cnt, 1.0)
    gr_h = jnp.dot(onehot, h_ref[...], preferred_element_type=f32) * inv
    gr_c = jnp.dot(onehot, hc_ref[...], preferred_element_type=f32) * inv
    h1 = jnp.maximum(
        jnp.dot(gr_h, w1a[...], preferred_element_type=f32)
        + jnp.dot(gr_c, w1b[...], preferred_element_type=f32)
        + b1[...], 0.0)
    h2 = jnp.maximum(jnp.dot(h1, w2[...], preferred_element_type=f32)
                     + b2[...], 0.0)
    o_ref[...] = jax.nn.sigmoid(
        jnp.dot(h2, w3[...], preferred_element_type=f32) + b3[...])


_pool = pl.pallas_call(
    _pool_body,
    in_specs=[
        pl.BlockSpec((_N, _OUT), lambda: (0, 0)),
        pl.BlockSpec((_N, _OUT), lambda: (0, 0)),
        pl.BlockSpec((1, _N), lambda: (0, 0)),
        pl.BlockSpec((_OUT, 256), lambda: (0, 0)),
        pl.BlockSpec((_OUT, 256), lambda: (0, 0)),
        pl.BlockSpec((1, 256), lambda: (0, 0)),
        pl.BlockSpec((256, 128), lambda: (0, 0)),
        pl.BlockSpec((1, 128), lambda: (0, 0)),
        pl.BlockSpec((128, 1), lambda: (0, 0)),
        pl.BlockSpec((1, 1), lambda: (0, 0)),
    ],
    out_specs=pl.BlockSpec((_G, 1), lambda: (0, 0)),
    out_shape=jax.ShapeDtypeStruct((_G, 1), jnp.float32),
)


def kernel(x, edge_index, batch, W_enc, b_enc, weight, W_ih, W_hh,
           b_ih, b_hh, W1, b1, W2, b2, W3, b3):
    f32 = jnp.float32
    src = edge_index[0]
    dst = edge_index[1]
    zc = jnp.zeros((_ZR, _H), f32)

    W_encp = jnp.pad(W_enc, ((0, 0), (0, _OUT - _C)))
    b_encp = jnp.pad(b_enc, (0, _OUT - _C)).reshape(1, _OUT)
    wpad = jnp.pad(weight, ((0, 0), (0, 0), (0, _F - _OUT)))
    ihT = W_ih.T
    hhT = W_hh.T
    pad_f = ((0, _F - _OUT), (0, 0))
    wir = jnp.pad(ihT[:, 0 * _OUT:1 * _OUT], pad_f)
    wiz = jnp.pad(ihT[:, 1 * _OUT:2 * _OUT], pad_f)
    win = jnp.pad(ihT[:, 2 * _OUT:3 * _OUT], pad_f)
    whr = hhT[:, 0 * _OUT:1 * _OUT]
    whz = hhT[:, 1 * _OUT:2 * _OUT]
    whn = hhT[:, 2 * _OUT:3 * _OUT]
    bir = b_ih[0 * _OUT:1 * _OUT].reshape(1, _OUT)
    biz = b_ih[1 * _OUT:2 * _OUT].reshape(1, _OUT)
    bin_ = b_ih[2 * _OUT:3 * _OUT].reshape(1, _OUT)
    bhr = b_hh[0 * _OUT:1 * _OUT].reshape(1, _OUT)
    bhz = b_hh[1 * _OUT:2 * _OUT].reshape(1, _OUT)
    bhn = b_hh[2 * _OUT:3 * _OUT].reshape(1, _OUT)
    W1a = W1[:_OUT]
    W1b = jnp.pad(W1[_OUT:], ((0, 2 * _OUT - W1.shape[0]), (0, 0)))
    b1r = b1.reshape(1, 256)
    b2r = b2.reshape(1, 128)
    b3r = b3.reshape(1, 1)
    batch2 = batch.reshape(1, _N)

    h_enc, m = _encode(x, W_encp, b_encp, wpad[0])
    h = h_enc
    for i in range(_L):
        agg2 = _segsum(m.reshape(2 * _N, _H), src, dst, zc)
        h, m = _gru(agg2, h, wir, wiz, win, whr, whz, whn,
                    bir, biz, bin_, bhr, bhz, bhn, wpad[(i + 1) % _L])
    return _pool(h, h_enc, batch2, W1a, W1b, b1r, W2, b2r, W3, b3r)
